# Initial kernel scaffold; baseline (speedup 1.0000x reference)
import jax
import jax.numpy as jnp
from jax.experimental import pallas as pl


def kernel(content_feature, style_feature):
    raise NotImplementedError("write your pallas kernel here")


# speedup vs baseline: 1.0029x; 1.0029x over previous
import jax
import jax.numpy as jnp
import numpy as np
from jax.experimental import pallas as pl

K = 3
PAD = 1
EPS = 1e-8


def _whiten_x(x):
    B, C, H, W = x.shape
    f = x.reshape(B, C, H * W)
    f = f - f.mean(-1, keepdims=True)
    cov = jnp.einsum('bcn,bdn->bcd', f, f) / (H * W - 1)
    e, v = jnp.linalg.eigh(cov)
    d = 1.0 / jnp.sqrt(jnp.maximum(e, EPS))
    wmat = jnp.einsum('bce,be,bde->bcd', v, d, v)
    return jnp.einsum('bcd,bdn->bcn', wmat, f).reshape(B, C, H, W)


def _color_x(x, target):
    B, C, H, W = x.shape
    f = x.reshape(B, C, H * W)
    t = target.reshape(B, C, -1)
    tm = t.mean(-1, keepdims=True)
    tc = t - tm
    cov = jnp.einsum('bcn,bdn->bcd', tc, tc) / (tc.shape[-1] - 1)
    e, v = jnp.linalg.eigh(cov)
    d = jnp.sqrt(jnp.maximum(e, 0.0))
    cmat = jnp.einsum('bce,be,bde->bcd', v, d, v)
    out = jnp.einsum('bcd,bdn->bcn', cmat, f) + tm
    return out.reshape(B, C, H, W)


def _extract_patches_x(x):
    B, C, H, W = x.shape
    xp = jnp.pad(x, ((0, 0), (0, 0), (PAD, PAD), (PAD, PAD)))
    pats = jnp.stack([xp[:, :, i:i + H, j:j + W] for i in range(K) for j in range(K)], axis=-1)
    return pats.reshape(B, C, H * W, K * K).transpose(0, 2, 1, 3)


def _decorate(content_feature, style_feature):
    nc = _whiten_x(content_feature)
    ns = _whiten_x(style_feature)
    B, C, H, W = nc.shape
    P = ns.shape[2] * ns.shape[3]
    kern = _extract_patches_x(ns).reshape(B, P, C * K * K)
    knorm = jnp.linalg.norm(kern, axis=2, keepdims=True) + 1e-5
    cpat = _extract_patches_x(nc).reshape(B, H * W, C * K * K)
    kn = (kern / knorm).astype(jnp.bfloat16)
    score = jnp.einsum('bqm,bpm->bpq', cpat.astype(jnp.bfloat16), kn,
                       preferred_element_type=jnp.float32)
    idx = jnp.argmax(score, axis=1)
    gathered = jnp.take_along_axis(kern, idx[:, :, None], axis=1)
    G = gathered.reshape(B, H, W, C, K, K).transpose(0, 3, 1, 2, 4, 5)
    full = jnp.zeros((B, C, H + K - 1, W + K - 1), dtype=nc.dtype)
    for dx in range(K):
        for dy in range(K):
            full = full.at[:, :, dx:dx + H, dy:dy + W].add(G[..., dx, dy])
    out = full[:, :, PAD:PAD + H, PAD:PAD + W]
    nfull = np.zeros((H + K - 1, W + K - 1), dtype=np.float32)
    for dx in range(K):
        for dy in range(K):
            nfull[dx:dx + H, dy:dy + W] += 1.0
    dn = jnp.asarray(nfull[PAD:PAD + H, PAD:PAD + W])
    reassembled = out / dn
    stylized = _color_x(reassembled, style_feature)
    return stylized


def kernel(content_feature, style_feature):
    return _decorate(content_feature, style_feature)


# speedup vs baseline: 1.1992x; 1.1957x over previous
import jax
import jax.numpy as jnp
import numpy as np
from jax.experimental import pallas as pl
from jax.experimental.pallas import tpu as pltpu

_KS = 3
_PAD = 1
_EPS = 1e-8

_C = 512
_H = 64
_W = 64
_HW = _H * _W
_CK = _C * _KS * _KS
_QT = 512
_PT = 512
_NQ = _HW // _QT
_NP = _HW // _PT


def _whiten(x):
    B, C, H, W = x.shape
    f = x.reshape(B, C, H * W)
    f = f - f.mean(-1, keepdims=True)
    cov = jnp.einsum('bcn,bdn->bcd', f, f) / (H * W - 1)
    e, v = jnp.linalg.eigh(cov)
    d = 1.0 / jnp.sqrt(jnp.maximum(e, _EPS))
    wmat = jnp.einsum('bce,be,bde->bcd', v, d, v)
    return jnp.einsum('bcd,bdn->bcn', wmat, f).reshape(B, C, H, W)


def _color(x, target):
    B, C, H, W = x.shape
    f = x.reshape(B, C, H * W)
    t = target.reshape(B, C, -1)
    tm = t.mean(-1, keepdims=True)
    tc = t - tm
    cov = jnp.einsum('bcn,bdn->bcd', tc, tc) / (tc.shape[-1] - 1)
    e, v = jnp.linalg.eigh(cov)
    d = jnp.sqrt(jnp.maximum(e, 0.0))
    cmat = jnp.einsum('bce,be,bde->bcd', v, d, v)
    out = jnp.einsum('bcd,bdn->bcn', cmat, f) + tm
    return out.reshape(B, C, H, W)


def _shifted_stack(x, axis):
    B, C, H, W = x.shape
    xp = jnp.pad(x, ((0, 0), (0, 0), (_PAD, _PAD), (_PAD, _PAD)))
    slabs = [xp[:, :, i:i + H, j:j + W].reshape(B, C, H * W)
             for i in range(_KS) for j in range(_KS)]
    return jnp.stack(slabs, axis=axis)


def _score_body(kn_ref, ct_ref, idx_ref, rmax_ref, ridx_ref):
    pi = pl.program_id(2)
    s = jax.lax.dot_general(kn_ref[0], ct_ref[0], (((1,), (0,)), ((), ())),
                            preferred_element_type=jnp.float32)
    tmax = jnp.max(s, axis=0, keepdims=True)
    ii = jax.lax.broadcasted_iota(jnp.int32, s.shape, 0)
    tidx = jnp.min(jnp.where(s == tmax, ii, _HW), axis=0,
                   keepdims=True) + pi * _PT

    @pl.when(pi == 0)
    def _init():
        rmax_ref[...] = tmax
        ridx_ref[...] = tidx

    @pl.when(pi != 0)
    def _update():
        better = tmax > rmax_ref[...]
        rmax_ref[...] = jnp.where(better, tmax, rmax_ref[...])
        ridx_ref[...] = jnp.where(better, tidx, ridx_ref[...])

    @pl.when(pi == _NP - 1)
    def _emit():
        idx_ref[0, 0] = ridx_ref[...]


def _recon_body(idxp_ref, ks_ref, out_ref, acc_ref):
    yi = pl.program_id(1)
    pi = pl.program_id(2)
    win = idxp_ref[0, :, pl.ds(pl.multiple_of(yi * _QT, 128), 768)]

    lane = jax.lax.broadcasted_iota(jnp.int32, (1, _QT), 1)
    x = lane & 63
    yg = (lane >> 6) + yi * 8
    pio = jax.lax.broadcasted_iota(jnp.int32, (_PT, _QT), 0) + pi * _PT

    tot = jnp.zeros((_C, _QT), jnp.float32)
    for dx in range(_KS):
        for dy in range(_KS):
            sh = (3 - dx) * 64 + (1 - dy)
            idxs = win[:, sh:sh + _QT]
            yv = (yg >= dx - 1) & (yg <= 62 + dx)
            xv = (x >= dy - 1) & (x <= 62 + dy)
            oh = jnp.where((idxs == pio) & yv & xv, 1.0, 0.0)
            oh = oh.astype(jnp.bfloat16)
            tot = tot + jax.lax.dot_general(
                ks_ref[0, dx * 3 + dy], oh, (((1,), (0,)), ((), ())),
                preferred_element_type=jnp.float32)

    @pl.when(pi == 0)
    def _init():
        acc_ref[...] = tot

    @pl.when(pi != 0)
    def _add():
        acc_ref[...] = acc_ref[...] + tot

    @pl.when(pi == _NP - 1)
    def _emit():
        cy = jnp.where((yg == 0) | (yg == 63), 2.0, 3.0)
        cx = jnp.where((x == 0) | (x == 63), 2.0, 3.0)
        out_ref[0] = acc_ref[...] / (cy * cx)


def _patch_swap(nc, ns):
    B = nc.shape[0]

    kern = _shifted_stack(ns, 2).reshape(B, _C, _KS * _KS, _HW)
    kern = kern.transpose(0, 3, 1, 2).reshape(B, _HW, _CK)
    knorm = jnp.linalg.norm(kern, axis=2, keepdims=True) + 1e-5
    kn = (kern / knorm).astype(jnp.bfloat16)

    ct = _shifted_stack(nc, 2).reshape(B, _CK, _HW).astype(jnp.bfloat16)

    idx4 = pl.pallas_call(
        _score_body,
        grid=(B, _NQ, _NP),
        in_specs=[
            pl.BlockSpec((1, _PT, _CK), lambda b, q, p: (b, p, 0)),
            pl.BlockSpec((1, _CK, _QT), lambda b, q, p: (b, 0, q)),
        ],
        out_specs=pl.BlockSpec((1, 1, 1, _QT), lambda b, q, p: (b, q, 0, 0)),
        out_shape=jax.ShapeDtypeStruct((B, _NQ, 1, _QT), jnp.int32),
        scratch_shapes=[pltpu.VMEM((1, _QT), jnp.float32),
                        pltpu.VMEM((1, _QT), jnp.int32)],
        compiler_params=pltpu.CompilerParams(
            dimension_semantics=("parallel", "arbitrary", "arbitrary")),
    )(kn, ct)

    idx = idx4.reshape(B, _H, _W)
    idxp = jnp.pad(idx, ((0, 0), (2, 2), (0, 0))).reshape(B, 1, 68 * 64)

    ks = _shifted_stack(ns, 1).astype(jnp.bfloat16)

    out_flat = pl.pallas_call(
        _recon_body,
        grid=(B, _NQ, _NP),
        in_specs=[
            pl.BlockSpec((1, 1, 68 * 64), lambda b, y, p: (b, 0, 0)),
            pl.BlockSpec((1, 9, _C, _PT), lambda b, y, p: (b, 0, 0, p)),
        ],
        out_specs=pl.BlockSpec((1, _C, _QT), lambda b, y, p: (b, 0, y)),
        out_shape=jax.ShapeDtypeStruct((B, _C, _HW), jnp.float32),
        scratch_shapes=[pltpu.VMEM((_C, _QT), jnp.float32)],
        compiler_params=pltpu.CompilerParams(
            dimension_semantics=("parallel", "arbitrary", "arbitrary")),
    )(idxp, ks)

    return out_flat.reshape(B, _C, _H, _W)


def kernel(content_feature, style_feature):
    nc = _whiten(content_feature)
    ns = _whiten(style_feature)
    reassembled = _patch_swap(nc, ns)
    return _color(reassembled, style_feature)
